# baseline (speedup 1.0000x reference)
import functools

import jax
import jax.numpy as jnp
from jax import lax
from jax.experimental import pallas as pl
from jax.experimental.pallas import tpu as pltpu
from jax.experimental.pallas import tpu_sc as plsc

_ROWS = 9216
_D = 128
_V = 4096
_BETA = 0.25
_BR = 512
_NSTEPS = _ROWS // _BR
_LC = 128

_NW = 32
_NCH = 3
_CH = 96
_BPW = _NCH * _CH


def _tc_body(x_ref, cbt_ref, xsq_ref, csq_ref, tok_ref, loss_ref):
    i = pl.program_id(0)

    @pl.when(i == 0)
    def _init():
        loss_ref[...] = jnp.zeros_like(loss_ref)

    x = x_ref[...]
    mm = jnp.dot(x, cbt_ref[...],
                 preferred_element_type=jnp.float32)
    xsq = xsq_ref[...]
    csq = csq_ref[...]

    lane = lax.broadcasted_iota(jnp.int32, (_BR, _LC), 1)
    bval = jnp.full((_BR, _LC), jnp.inf, jnp.float32)
    bidx = jnp.zeros((_BR, _LC), jnp.int32)
    for t in range(_V // _LC):
        lo = t * _LC
        d2 = (xsq + csq[:, lo:lo + _LC]) - 2.0 * mm[:, lo:lo + _LC]
        dist = jnp.sqrt(jnp.maximum(d2, 0.0))
        better = dist < bval
        bval = jnp.where(better, dist, bval)
        bidx = jnp.where(better, lane + lo, bidx)

    for sh in (64, 32, 16, 8, 4, 2, 1):
        v2 = pltpu.roll(bval, sh, 1)
        i2 = pltpu.roll(bidx, sh, 1)
        take = (v2 < bval) | ((v2 == bval) & (i2 < bidx))
        bval = jnp.where(take, v2, bval)
        bidx = jnp.where(take, i2, bidx)

    tok_ref[...] = bidx[:, 0].reshape(1, 1, _BR)

    dmin = bval[:, 0:1]
    loss_ref[...] += jnp.sum(dmin * dmin).reshape(1, 1)

    @pl.when(i == _NSTEPS - 1)
    def _fin():
        loss_ref[...] = loss_ref[...] * ((1.0 + _BETA) / (_ROWS * _D))


def _tc_call(xf, cbt, xsq, csq):
    return pl.pallas_call(
        _tc_body,
        grid=(_NSTEPS,),
        in_specs=[
            pl.BlockSpec((_BR, _D), lambda i: (i, 0)),
            pl.BlockSpec((_D, _V), lambda i: (0, 0)),
            pl.BlockSpec((_BR, 1), lambda i: (i, 0)),
            pl.BlockSpec((1, _V), lambda i: (0, 0)),
        ],
        out_specs=[
            pl.BlockSpec((1, 1, _BR), lambda i: (i, 0, 0)),
            pl.BlockSpec((1, 1), lambda i: (0, 0)),
        ],
        out_shape=[
            jax.ShapeDtypeStruct((_NSTEPS, 1, _BR), jnp.int32),
            jax.ShapeDtypeStruct((1, 1), jnp.float32),
        ],
        compiler_params=pltpu.CompilerParams(
            dimension_semantics=("arbitrary",)),
    )(xf, cbt, xsq, csq)


def _sc_gather_body(cb_hbm, tok_hbm, out_hbm, idx_v, rows_v, sem):
    c = lax.axis_index("c")
    s = lax.axis_index("s")
    wid = s * 2 + c
    base = wid * _BPW
    pltpu.sync_copy(tok_hbm.at[pl.ds(base, _BPW)], idx_v)
    copies = [
        pltpu.async_copy(cb_hbm.at[idx_v.at[pl.ds(j * _CH, _CH)]],
                         rows_v.at[pl.ds(j * _CH, _CH)], sem)
        for j in range(_NCH)
    ]
    for cp in copies:
        cp.wait()
    pltpu.sync_copy(rows_v, out_hbm.at[pl.ds(base, _BPW)])


def _sc_gather(codebook, tok_flat):
    mesh = plsc.VectorSubcoreMesh(core_axis_name="c", subcore_axis_name="s")
    k = functools.partial(
        pl.kernel,
        mesh=mesh,
        out_type=jax.ShapeDtypeStruct((_ROWS, _D), jnp.float32),
        scratch_types=[
            pltpu.VMEM((_BPW,), jnp.int32),
            pltpu.VMEM((_BPW, _D), jnp.float32),
            pltpu.SemaphoreType.DMA,
        ],
    )(_sc_gather_body)
    return k(codebook, tok_flat)


def kernel(x, codebook):
    B, N, D = x.shape
    xf = x.reshape(-1, D)
    xsq = jnp.sum(xf * xf, axis=1, keepdims=True)
    csq = jnp.sum(codebook * codebook, axis=1)[None, :]
    tok3, loss = _tc_call(xf, codebook.T, xsq, csq)
    tokens_flat = tok3.reshape(-1)
    q = _sc_gather(codebook, tokens_flat)
    tokens = tokens_flat.reshape(B, N)
    quantized_st = q.reshape(B, N, D)
    return tokens, quantized_st, loss.reshape(())


# speedup vs baseline: 1.0250x; 1.0250x over previous
import functools

import jax
import jax.numpy as jnp
from jax import lax
from jax.experimental import pallas as pl
from jax.experimental.pallas import tpu as pltpu
from jax.experimental.pallas import tpu_sc as plsc

_ROWS = 9216
_D = 128
_V = 4096
_BETA = 0.25
_BR = 512
_NSTEPS = _ROWS // _BR

_NW = 32
_NCH = 3
_CH = 96
_BPW = _NCH * _CH


def _tc_body(x_ref, cbt_ref, xsq_ref, csq_ref, tok_ref, loss_ref):
    i = pl.program_id(0)

    @pl.when(i == 0)
    def _init():
        loss_ref[...] = jnp.zeros_like(loss_ref)

    nmm = jnp.dot(x_ref[...], cbt_ref[...],
                  preferred_element_type=jnp.float32)
    d2 = xsq_ref[...] + csq_ref[...] + nmm

    dist = jnp.sqrt(jnp.maximum(d2, 0.0))
    dmin = jnp.min(dist, axis=1, keepdims=True)
    idx = lax.broadcasted_iota(jnp.int32, (1, _V), 1)
    tok = jnp.min(jnp.where(dist == dmin, idx, _V), axis=1)
    tok_ref[...] = tok.reshape(1, 1, _BR)

    loss_ref[...] += jnp.sum(dmin * dmin).reshape(1, 1)

    @pl.when(i == _NSTEPS - 1)
    def _fin():
        loss_ref[...] = loss_ref[...] * ((1.0 + _BETA) / (_ROWS * _D))


def _tc_call(xf, cbt, xsq, csq):
    return pl.pallas_call(
        _tc_body,
        grid=(_NSTEPS,),
        in_specs=[
            pl.BlockSpec((_BR, _D), lambda i: (i, 0)),
            pl.BlockSpec((_D, _V), lambda i: (0, 0)),
            pl.BlockSpec((_BR, 1), lambda i: (i, 0)),
            pl.BlockSpec((1, _V), lambda i: (0, 0)),
        ],
        out_specs=[
            pl.BlockSpec((1, 1, _BR), lambda i: (i, 0, 0)),
            pl.BlockSpec((1, 1), lambda i: (0, 0)),
        ],
        out_shape=[
            jax.ShapeDtypeStruct((_NSTEPS, 1, _BR), jnp.int32),
            jax.ShapeDtypeStruct((1, 1), jnp.float32),
        ],
        compiler_params=pltpu.CompilerParams(
            dimension_semantics=("arbitrary",)),
    )(xf, cbt, xsq, csq)


def _sc_gather_body(cb_hbm, tok_hbm, out_hbm, idx_v, rows_v, sem):
    c = lax.axis_index("c")
    s = lax.axis_index("s")
    wid = s * 2 + c
    base = wid * _BPW
    pltpu.sync_copy(tok_hbm.at[pl.ds(base, _BPW)], idx_v)
    copies = [
        pltpu.async_copy(cb_hbm.at[idx_v.at[pl.ds(j * _CH, _CH)]],
                         rows_v.at[pl.ds(j * _CH, _CH)], sem)
        for j in range(_NCH)
    ]
    for cp in copies:
        cp.wait()
    pltpu.sync_copy(rows_v, out_hbm.at[pl.ds(base, _BPW)])


def _sc_gather(codebook, tok_flat):
    mesh = plsc.VectorSubcoreMesh(core_axis_name="c", subcore_axis_name="s")
    k = functools.partial(
        pl.kernel,
        mesh=mesh,
        out_type=jax.ShapeDtypeStruct((_ROWS, _D), jnp.float32),
        scratch_types=[
            pltpu.VMEM((_BPW,), jnp.int32),
            pltpu.VMEM((_BPW, _D), jnp.float32),
            pltpu.SemaphoreType.DMA,
        ],
    )(_sc_gather_body)
    return k(codebook, tok_flat)


def kernel(x, codebook):
    B, N, D = x.shape
    xf = x.reshape(-1, D)
    xsq = jnp.sum(xf * xf, axis=1, keepdims=True)
    csq = jnp.sum(codebook * codebook, axis=1)[None, :]
    tok3, loss = _tc_call(-2.0 * xf, codebook.T, xsq, csq)
    tokens_flat = tok3.reshape(-1)
    q = _sc_gather(codebook, tokens_flat)
    tokens = tokens_flat.reshape(B, N)
    quantized_st = q.reshape(B, N, D)
    return tokens, quantized_st, loss.reshape(())


# speedup vs baseline: 1.1277x; 1.1003x over previous
import functools

import jax
import jax.numpy as jnp
from jax import lax
from jax.experimental import pallas as pl
from jax.experimental.pallas import tpu as pltpu
from jax.experimental.pallas import tpu_sc as plsc

_ROWS = 9216
_D = 128
_V = 4096
_BETA = 0.25
_BR = 1024
_NSTEPS = _ROWS // _BR

_NW = 32
_NCH = 3
_CH = 96
_BPW = _NCH * _CH


def _tc_body(x_ref, cbt_ref, xsq_ref, csq_ref, tok_ref, loss_ref):
    i = pl.program_id(0)

    @pl.when(i == 0)
    def _init():
        loss_ref[...] = jnp.zeros_like(loss_ref)

    nmm = jnp.dot(-2.0 * x_ref[...], cbt_ref[...],
                  preferred_element_type=jnp.float32)
    d2 = xsq_ref[...] + csq_ref[...] + nmm

    dist = jnp.sqrt(jnp.maximum(d2, 0.0))
    dmin = jnp.min(dist, axis=1, keepdims=True)
    idx = lax.broadcasted_iota(jnp.int32, (1, _V), 1)
    tok = jnp.min(jnp.where(dist == dmin, idx, _V), axis=1)
    tok_ref[...] = tok.reshape(1, 1, _BR)

    loss_ref[...] += jnp.sum(dmin * dmin).reshape(1, 1)

    @pl.when(i == _NSTEPS - 1)
    def _fin():
        loss_ref[...] = loss_ref[...] * ((1.0 + _BETA) / (_ROWS * _D))


def _tc_call(xf, cbt, xsq, csq):
    return pl.pallas_call(
        _tc_body,
        grid=(_NSTEPS,),
        in_specs=[
            pl.BlockSpec((_BR, _D), lambda i: (i, 0)),
            pl.BlockSpec((_D, _V), lambda i: (0, 0)),
            pl.BlockSpec((_BR, 1), lambda i: (i, 0)),
            pl.BlockSpec((1, _V), lambda i: (0, 0)),
        ],
        out_specs=[
            pl.BlockSpec((1, 1, _BR), lambda i: (i, 0, 0)),
            pl.BlockSpec((1, 1), lambda i: (0, 0)),
        ],
        out_shape=[
            jax.ShapeDtypeStruct((_NSTEPS, 1, _BR), jnp.int32),
            jax.ShapeDtypeStruct((1, 1), jnp.float32),
        ],
        compiler_params=pltpu.CompilerParams(
            dimension_semantics=("arbitrary",)),
    )(xf, cbt, xsq, csq)


def _sc_gather_body(cb_hbm, tok_hbm, out_hbm, idx_v, rows_v, sem):
    c = lax.axis_index("c")
    s = lax.axis_index("s")
    wid = s * 2 + c
    base = wid * _BPW
    pltpu.sync_copy(tok_hbm.at[pl.ds(base, _BPW)], idx_v)
    copies = [
        pltpu.async_copy(cb_hbm.at[idx_v.at[pl.ds(j * _CH, _CH)]],
                         rows_v.at[pl.ds(j * _CH, _CH)], sem)
        for j in range(_NCH)
    ]
    for cp in copies:
        cp.wait()
    pltpu.sync_copy(rows_v, out_hbm.at[pl.ds(base, _BPW)])


def _sc_gather(codebook, tok_flat):
    mesh = plsc.VectorSubcoreMesh(core_axis_name="c", subcore_axis_name="s")
    k = functools.partial(
        pl.kernel,
        mesh=mesh,
        out_type=jax.ShapeDtypeStruct((_ROWS, _D), jnp.float32),
        scratch_types=[
            pltpu.VMEM((_BPW,), jnp.int32),
            pltpu.VMEM((_BPW, _D), jnp.float32),
            pltpu.SemaphoreType.DMA,
        ],
    )(_sc_gather_body)
    return k(codebook, tok_flat)


def kernel(x, codebook):
    B, N, D = x.shape
    xf = x.reshape(-1, D)
    xsq = jnp.sum(xf * xf, axis=1, keepdims=True)
    csq = jnp.sum(codebook * codebook, axis=1)[None, :]
    tok3, loss = _tc_call(xf, codebook.T, xsq, csq)
    tokens_flat = tok3.reshape(-1)
    q = _sc_gather(codebook, tokens_flat)
    tokens = tokens_flat.reshape(B, N)
    quantized_st = q.reshape(B, N, D)
    return tokens, quantized_st, loss.reshape(())


# speedup vs baseline: 1.1527x; 1.0221x over previous
import functools

import jax
import jax.numpy as jnp
from jax import lax
from jax.experimental import pallas as pl
from jax.experimental.pallas import tpu as pltpu
from jax.experimental.pallas import tpu_sc as plsc

_ROWS = 9216
_D = 128
_V = 4096
_BETA = 0.25
_BR = 1536
_NSTEPS = _ROWS // _BR

_NW = 32
_NCH = 3
_CH = 96
_BPW = _NCH * _CH


def _tc_body(x_ref, cbt_ref, xsq_ref, csq_ref, tok_ref, loss_ref):
    i = pl.program_id(0)

    @pl.when(i == 0)
    def _init():
        loss_ref[...] = jnp.zeros_like(loss_ref)

    nmm = jnp.dot(-2.0 * x_ref[...], cbt_ref[...],
                  preferred_element_type=jnp.float32)
    d2 = xsq_ref[...] + csq_ref[...] + nmm

    dist = jnp.sqrt(jnp.maximum(d2, 0.0))
    dmin = jnp.min(dist, axis=1, keepdims=True)
    idx = lax.broadcasted_iota(jnp.int32, (1, _V), 1)
    tok = jnp.min(jnp.where(dist == dmin, idx, _V), axis=1)
    tok_ref[...] = tok.reshape(1, 1, _BR)

    loss_ref[...] += jnp.sum(dmin * dmin).reshape(1, 1)

    @pl.when(i == _NSTEPS - 1)
    def _fin():
        loss_ref[...] = loss_ref[...] * ((1.0 + _BETA) / (_ROWS * _D))


def _tc_call(xf, cbt, xsq, csq):
    return pl.pallas_call(
        _tc_body,
        grid=(_NSTEPS,),
        in_specs=[
            pl.BlockSpec((_BR, _D), lambda i: (i, 0)),
            pl.BlockSpec((_D, _V), lambda i: (0, 0)),
            pl.BlockSpec((_BR, 1), lambda i: (i, 0)),
            pl.BlockSpec((1, _V), lambda i: (0, 0)),
        ],
        out_specs=[
            pl.BlockSpec((1, 1, _BR), lambda i: (i, 0, 0)),
            pl.BlockSpec((1, 1), lambda i: (0, 0)),
        ],
        out_shape=[
            jax.ShapeDtypeStruct((_NSTEPS, 1, _BR), jnp.int32),
            jax.ShapeDtypeStruct((1, 1), jnp.float32),
        ],
        compiler_params=pltpu.CompilerParams(
            dimension_semantics=("arbitrary",)),
    )(xf, cbt, xsq, csq)


def _sc_gather_body(cb_hbm, tok_hbm, out_hbm, idx_v, rows_v, sem):
    c = lax.axis_index("c")
    s = lax.axis_index("s")
    wid = s * 2 + c
    base = wid * _BPW
    pltpu.sync_copy(tok_hbm.at[pl.ds(base, _BPW)], idx_v)
    copies = [
        pltpu.async_copy(cb_hbm.at[idx_v.at[pl.ds(j * _CH, _CH)]],
                         rows_v.at[pl.ds(j * _CH, _CH)], sem)
        for j in range(_NCH)
    ]
    for cp in copies:
        cp.wait()
    pltpu.sync_copy(rows_v, out_hbm.at[pl.ds(base, _BPW)])


def _sc_gather(codebook, tok_flat):
    mesh = plsc.VectorSubcoreMesh(core_axis_name="c", subcore_axis_name="s")
    k = functools.partial(
        pl.kernel,
        mesh=mesh,
        out_type=jax.ShapeDtypeStruct((_ROWS, _D), jnp.float32),
        scratch_types=[
            pltpu.VMEM((_BPW,), jnp.int32),
            pltpu.VMEM((_BPW, _D), jnp.float32),
            pltpu.SemaphoreType.DMA,
        ],
    )(_sc_gather_body)
    return k(codebook, tok_flat)


def kernel(x, codebook):
    B, N, D = x.shape
    xf = x.reshape(-1, D)
    xsq = jnp.sum(xf * xf, axis=1, keepdims=True)
    csq = jnp.sum(codebook * codebook, axis=1)[None, :]
    tok3, loss = _tc_call(xf, codebook.T, xsq, csq)
    tokens_flat = tok3.reshape(-1)
    q = _sc_gather(codebook, tokens_flat)
    tokens = tokens_flat.reshape(B, N)
    quantized_st = q.reshape(B, N, D)
    return tokens, quantized_st, loss.reshape(())
